# Initial kernel scaffold; baseline (speedup 1.0000x reference)
import jax
import jax.numpy as jnp
from jax.experimental import pallas as pl


def kernel(x, W0, b0, W1, b1, W2, b2, y):
    raise NotImplementedError("write your pallas kernel here")


# speedup vs baseline: 2.8485x; 2.8485x over previous
import functools

import jax
import jax.numpy as jnp
from jax import lax
from jax.experimental import pallas as pl
from jax.experimental.pallas import tpu as pltpu
from jax.experimental.pallas import tpu_sc as plsc

K = 8
IN_DIM = 1024
OUT_DIM = 1024
H0 = 2048
H1 = 2048
M = 8192
BLK = 256
PM = M + K * BLK
NB = PM // BLK
NC, NS = 2, 16
NW = NC * NS


def _route(y):
    y = y.astype(jnp.int32)
    oh = (y[:, None] == jnp.arange(K, dtype=jnp.int32)[None, :]).astype(jnp.int32)
    csum = jnp.cumsum(oh, axis=0)
    counts = csum[-1]
    rank = jnp.take_along_axis(csum, y[:, None], axis=1)[:, 0] - 1
    padded = ((counts + BLK - 1) // BLK) * BLK
    ends = jnp.cumsum(padded)
    starts = ends - padded
    pos = starts[y] + rank
    src = jnp.zeros((PM,), jnp.int32).at[pos].set(jnp.arange(M, dtype=jnp.int32))
    blk_starts = jnp.arange(NB, dtype=jnp.int32) * BLK
    be = jnp.minimum(jnp.searchsorted(ends, blk_starts, side='right'), K - 1)
    return src, pos, be.astype(jnp.int32)


def _make_gather(n_rows, d, dtype, chunk):
    rpw = n_rows // NW
    mesh = plsc.VectorSubcoreMesh(core_axis_name="c", subcore_axis_name="s")

    @functools.partial(
        pl.kernel, mesh=mesh,
        out_type=jax.ShapeDtypeStruct((n_rows, d), dtype),
        scratch_types=[
            pltpu.VMEM((chunk,), jnp.int32),
            pltpu.VMEM((chunk, d), dtype),
            pltpu.SemaphoreType.DMA,
        ],
    )
    def gk(table_hbm, idx_hbm, out_hbm, idx_v, rows_v, sem):
        wid = lax.axis_index("s") * NC + lax.axis_index("c")
        base0 = wid * rpw
        for c in range(rpw // chunk):
            base = base0 + c * chunk
            pltpu.sync_copy(idx_hbm.at[pl.ds(base, chunk)], idx_v)
            pltpu.async_copy(table_hbm.at[idx_v], rows_v, sem).wait()
            pltpu.sync_copy(rows_v, out_hbm.at[pl.ds(base, chunk)])

    return gk


def _make_gather2(n_rows, d, chunk):
    rpw = n_rows // NW
    mesh = plsc.VectorSubcoreMesh(core_axis_name="c", subcore_axis_name="s")

    @functools.partial(
        pl.kernel, mesh=mesh,
        out_type=(jax.ShapeDtypeStruct((n_rows, d), jnp.float32),
                  jax.ShapeDtypeStruct((n_rows, d), jnp.float32)),
        scratch_types=[
            pltpu.VMEM((chunk,), jnp.int32),
            pltpu.VMEM((chunk, d), jnp.float32),
            pltpu.SemaphoreType.DMA,
        ],
    )
    def gk(ta_hbm, tb_hbm, idx_hbm, oa_hbm, ob_hbm, idx_v, rows_v, sem):
        wid = lax.axis_index("s") * NC + lax.axis_index("c")
        base0 = wid * rpw
        for c in range(rpw // chunk):
            base = base0 + c * chunk
            pltpu.sync_copy(idx_hbm.at[pl.ds(base, chunk)], idx_v)
            pltpu.async_copy(ta_hbm.at[idx_v], rows_v, sem).wait()
            pltpu.sync_copy(rows_v, oa_hbm.at[pl.ds(base, chunk)])
            pltpu.async_copy(tb_hbm.at[idx_v], rows_v, sem).wait()
            pltpu.sync_copy(rows_v, ob_hbm.at[pl.ds(base, chunk)])

    return gk


_gather_x = _make_gather(PM, IN_DIM, jnp.float32, 64)
_gather_out = _make_gather2(M, OUT_DIM, 64)


def _l01_body(eb_ref, x_ref, w_ref, b_ref, o_ref):
    a = x_ref[...].astype(jnp.bfloat16)
    w = w_ref[0].astype(jnp.bfloat16)
    acc = jnp.dot(a, w, preferred_element_type=jnp.float32) + b_ref[0]
    o_ref[...] = jnp.maximum(acc, 0.0).astype(jnp.bfloat16)


def _l2_body(eb_ref, x_ref, w_ref, b_ref, mu_ref, sig_ref):
    a = x_ref[...]
    w = w_ref[0].astype(jnp.bfloat16)
    acc = jnp.dot(a, w, preferred_element_type=jnp.float32) + b_ref[0]
    mu_ref[...] = acc[:, :OUT_DIM]
    s = acc[:, OUT_DIM:]
    sig_ref[...] = jnp.maximum(s, 0.0) + jnp.log1p(jnp.exp(-jnp.abs(s)))


def _grouped_layer(x, W, b, be, din, dout):
    return pl.pallas_call(
        _l01_body,
        grid_spec=pltpu.PrefetchScalarGridSpec(
            num_scalar_prefetch=1,
            grid=(NB,),
            in_specs=[
                pl.BlockSpec((BLK, din), lambda i, eb: (i, 0)),
                pl.BlockSpec((1, din, dout), lambda i, eb: (eb[i], 0, 0)),
                pl.BlockSpec((1, 1, dout), lambda i, eb: (eb[i], 0, 0)),
            ],
            out_specs=pl.BlockSpec((BLK, dout), lambda i, eb: (i, 0)),
        ),
        out_shape=jax.ShapeDtypeStruct((PM, dout), jnp.bfloat16),
    )(be, x, W, b.reshape(K, 1, dout))


def _final_layer(x, W, b, be, din):
    return pl.pallas_call(
        _l2_body,
        grid_spec=pltpu.PrefetchScalarGridSpec(
            num_scalar_prefetch=1,
            grid=(NB,),
            in_specs=[
                pl.BlockSpec((BLK, din), lambda i, eb: (i, 0)),
                pl.BlockSpec((1, din, 2 * OUT_DIM), lambda i, eb: (eb[i], 0, 0)),
                pl.BlockSpec((1, 1, 2 * OUT_DIM), lambda i, eb: (eb[i], 0, 0)),
            ],
            out_specs=(
                pl.BlockSpec((BLK, OUT_DIM), lambda i, eb: (i, 0)),
                pl.BlockSpec((BLK, OUT_DIM), lambda i, eb: (i, 0)),
            ),
        ),
        out_shape=(
            jax.ShapeDtypeStruct((PM, OUT_DIM), jnp.float32),
            jax.ShapeDtypeStruct((PM, OUT_DIM), jnp.float32),
        ),
    )(be, x, W, b.reshape(K, 1, 2 * OUT_DIM))


def kernel(x, W0, b0, W1, b1, W2, b2, y):
    src, pos, be = _route(y)
    xs = _gather_x(x, src)
    h = _grouped_layer(xs, W0, b0, be, IN_DIM, H0)
    h = _grouped_layer(h, W1, b1, be, H0, H1)
    mu_s, sig_s = _final_layer(h, W2, b2, be, H1)
    mu, sigma = _gather_out(mu_s, sig_s, pos)
    return mu, sigma


# speedup vs baseline: 3.4694x; 1.2180x over previous
import functools

import jax
import jax.numpy as jnp
from jax import lax
from jax.experimental import pallas as pl
from jax.experimental.pallas import tpu as pltpu
from jax.experimental.pallas import tpu_sc as plsc

K = 8
IN_DIM = 1024
OUT_DIM = 1024
H0 = 2048
H1 = 2048
M = 8192
BLK = 256
PM = M + K * BLK
NB = PM // BLK
NC, NS = 2, 16
NW = NC * NS


def _route(y):
    y = y.astype(jnp.int32)
    oh = (y[:, None] == jnp.arange(K, dtype=jnp.int32)[None, :]).astype(jnp.int32)
    csum = jnp.cumsum(oh, axis=0)
    counts = csum[-1]
    rank = jnp.take_along_axis(csum, y[:, None], axis=1)[:, 0] - 1
    padded = ((counts + BLK - 1) // BLK) * BLK
    ends = jnp.cumsum(padded)
    starts = ends - padded
    pos = starts[y] + rank
    src = jnp.mod(jnp.arange(PM, dtype=jnp.int32), M).at[pos].set(
        jnp.arange(M, dtype=jnp.int32))
    blk_starts = jnp.arange(NB, dtype=jnp.int32) * BLK
    be = jnp.minimum(jnp.searchsorted(ends, blk_starts, side='right'), K - 1)
    return src, pos, be.astype(jnp.int32)


_CHUNK = 32


def _make_gather(n_rows, d, dtype, chunk):
    rpw = n_rows // NW
    nch = rpw // chunk
    mesh = plsc.VectorSubcoreMesh(core_axis_name="c", subcore_axis_name="s")

    @functools.partial(
        pl.kernel, mesh=mesh,
        out_type=jax.ShapeDtypeStruct((n_rows, d), dtype),
        scratch_types=[
            pltpu.VMEM((nch, chunk), jnp.int32),
            pltpu.VMEM((chunk, d), dtype),
            pltpu.VMEM((chunk, d), dtype),
            pltpu.SemaphoreType.DMA,
            pltpu.SemaphoreType.DMA,
            pltpu.SemaphoreType.DMA,
            pltpu.SemaphoreType.DMA,
        ],
    )
    def gk(table_hbm, idx_hbm, out_hbm, idx_v, b0, b1, gs0, gs1, os0, os1):
        wid = lax.axis_index("s") * NC + lax.axis_index("c")
        base0 = wid * rpw
        pltpu.sync_copy(idx_hbm.at[wid], idx_v)
        bufs, gsems, osems = (b0, b1), (gs0, gs1), (os0, os1)
        gc = [None] * nch
        oc = [None] * nch
        for c in range(nch):
            s = c & 1
            if c >= 2:
                oc[c - 2].wait()
            gc[c] = pltpu.async_copy(table_hbm.at[idx_v.at[c]], bufs[s], gsems[s])
            if c >= 1:
                p = (c - 1) & 1
                gc[c - 1].wait()
                oc[c - 1] = pltpu.async_copy(
                    bufs[p], out_hbm.at[pl.ds(base0 + (c - 1) * chunk, chunk)],
                    osems[p])
        last = nch - 1
        gc[last].wait()
        oc[last] = pltpu.async_copy(
            bufs[last & 1], out_hbm.at[pl.ds(base0 + last * chunk, chunk)],
            osems[last & 1])
        if nch >= 2:
            oc[last - 1].wait()
        oc[last].wait()

    return gk


def _make_gather2(n_rows, d, chunk):
    rpw = n_rows // NW
    nch = rpw // chunk
    nst = 2 * nch
    mesh = plsc.VectorSubcoreMesh(core_axis_name="c", subcore_axis_name="s")

    @functools.partial(
        pl.kernel, mesh=mesh,
        out_type=(jax.ShapeDtypeStruct((n_rows, d), jnp.float32),
                  jax.ShapeDtypeStruct((n_rows, d), jnp.float32)),
        scratch_types=[
            pltpu.VMEM((nch, chunk), jnp.int32),
            pltpu.VMEM((chunk, d), jnp.float32),
            pltpu.VMEM((chunk, d), jnp.float32),
            pltpu.SemaphoreType.DMA,
            pltpu.SemaphoreType.DMA,
            pltpu.SemaphoreType.DMA,
            pltpu.SemaphoreType.DMA,
        ],
    )
    def gk(ta_hbm, tb_hbm, idx_hbm, oa_hbm, ob_hbm,
           idx_v, b0, b1, gs0, gs1, os0, os1):
        wid = lax.axis_index("s") * NC + lax.axis_index("c")
        base0 = wid * rpw
        pltpu.sync_copy(idx_hbm.at[wid], idx_v)
        bufs, gsems, osems = (b0, b1), (gs0, gs1), (os0, os1)
        tabs = (ta_hbm, tb_hbm)
        outs = (oa_hbm, ob_hbm)
        gc = [None] * nst
        oc = [None] * nst

        def _write(t):
            c, j = t // 2, t & 1
            return pltpu.async_copy(
                bufs[t & 1], outs[j].at[pl.ds(base0 + c * chunk, chunk)],
                osems[t & 1])

        for t in range(nst):
            c, j = t // 2, t & 1
            s = t & 1
            if t >= 2:
                oc[t - 2].wait()
            gc[t] = pltpu.async_copy(tabs[j].at[idx_v.at[c]], bufs[s], gsems[s])
            if t >= 1:
                gc[t - 1].wait()
                oc[t - 1] = _write(t - 1)
        gc[nst - 1].wait()
        oc[nst - 1] = _write(nst - 1)
        oc[nst - 2].wait()
        oc[nst - 1].wait()

    return gk


_gather_x = _make_gather(PM, IN_DIM, jnp.float32, _CHUNK)
_gather_out = _make_gather2(M, OUT_DIM, _CHUNK)


def _l01_body(eb_ref, x_ref, w_ref, b_ref, o_ref):
    a = x_ref[...].astype(jnp.bfloat16)
    w = w_ref[0].astype(jnp.bfloat16)
    acc = jnp.dot(a, w, preferred_element_type=jnp.float32) + b_ref[0]
    o_ref[...] = jnp.maximum(acc, 0.0).astype(jnp.bfloat16)


def _l2_body(eb_ref, x_ref, w_ref, b_ref, mu_ref, sig_ref):
    a = x_ref[...]
    w = w_ref[0].astype(jnp.bfloat16)
    acc = jnp.dot(a, w, preferred_element_type=jnp.float32) + b_ref[0]
    mu_ref[...] = acc[:, :OUT_DIM]
    s = acc[:, OUT_DIM:]
    sig_ref[...] = jnp.maximum(s, 0.0) + jnp.log1p(jnp.exp(-jnp.abs(s)))


def _grouped_layer(x, W, b, be, din, dout):
    return pl.pallas_call(
        _l01_body,
        grid_spec=pltpu.PrefetchScalarGridSpec(
            num_scalar_prefetch=1,
            grid=(NB,),
            in_specs=[
                pl.BlockSpec((BLK, din), lambda i, eb: (i, 0)),
                pl.BlockSpec((1, din, dout), lambda i, eb: (eb[i], 0, 0)),
                pl.BlockSpec((1, 1, dout), lambda i, eb: (eb[i], 0, 0)),
            ],
            out_specs=pl.BlockSpec((BLK, dout), lambda i, eb: (i, 0)),
        ),
        out_shape=jax.ShapeDtypeStruct((PM, dout), jnp.bfloat16),
    )(be, x, W, b.reshape(K, 1, dout))


def _final_layer(x, W, b, be, din):
    return pl.pallas_call(
        _l2_body,
        grid_spec=pltpu.PrefetchScalarGridSpec(
            num_scalar_prefetch=1,
            grid=(NB,),
            in_specs=[
                pl.BlockSpec((BLK, din), lambda i, eb: (i, 0)),
                pl.BlockSpec((1, din, 2 * OUT_DIM), lambda i, eb: (eb[i], 0, 0)),
                pl.BlockSpec((1, 1, 2 * OUT_DIM), lambda i, eb: (eb[i], 0, 0)),
            ],
            out_specs=(
                pl.BlockSpec((BLK, OUT_DIM), lambda i, eb: (i, 0)),
                pl.BlockSpec((BLK, OUT_DIM), lambda i, eb: (i, 0)),
            ),
        ),
        out_shape=(
            jax.ShapeDtypeStruct((PM, OUT_DIM), jnp.float32),
            jax.ShapeDtypeStruct((PM, OUT_DIM), jnp.float32),
        ),
    )(be, x, W, b.reshape(K, 1, 2 * OUT_DIM))


def kernel(x, W0, b0, W1, b1, W2, b2, y):
    src, pos, be = _route(y)
    src = src.reshape(NW, PM // NW // _CHUNK, _CHUNK)
    pos = pos.reshape(NW, M // NW // _CHUNK, _CHUNK)
    xs = _gather_x(x, src)
    h = _grouped_layer(xs, W0, b0, be, IN_DIM, H0)
    h = _grouped_layer(h, W1, b1, be, H0, H1)
    mu_s, sig_s = _final_layer(h, W2, b2, be, H1)
    mu, sigma = _gather_out(mu_s, sig_s, pos)
    return mu, sigma
